# baseline (speedup 1.0000x reference)
import functools

import jax
import jax.numpy as jnp
from jax import lax
from jax.experimental import pallas as pl
from jax.experimental.pallas import tpu as pltpu
from jax.experimental.pallas import tpu_sc as plsc

B = 16384
D = 64
NC = 2
NS = 16
NW = NC * NS
BW = B // NW
GSZ = 8
NP = 4
NG = BW // GSZ


def _transe_body(ent_hbm, rel_hbm, heads_hbm, rels_hbm, tails_hbm, out_hbm,
                 hvm, rvm, tvm, sh, sr, st, outv, tmp,
                 sem0, sem1, sem2, sem3):
    wid = lax.axis_index("s") * NC + lax.axis_index("c")
    base = wid * BW

    pltpu.sync_copy(heads_hbm.at[pl.ds(base, BW)], hvm)
    pltpu.sync_copy(rels_hbm.at[pl.ds(base, BW)], rvm)
    pltpu.sync_copy(tails_hbm.at[pl.ds(base, BW)], tvm)

    sems = (sem0, sem1, sem2, sem3)
    lanes = lax.iota(jnp.int32, 16)

    def issue_group(g, par):
        sem = sems[par]
        vbase = pl.multiple_of((g // 2) * 16, 16)
        hvec = hvm[pl.ds(vbase, 16)]
        rvec = rvm[pl.ds(vbase, 16)]
        tvec = tvm[pl.ds(vbase, 16)]
        odd = lax.rem(g, 2)
        for jj in range(GSZ):
            rb = (par * GSZ + jj) * 8
            hs = lax.select(odd == 1, hvec[GSZ + jj], hvec[jj])
            rs = lax.select(odd == 1, rvec[GSZ + jj], rvec[jj])
            ts = lax.select(odd == 1, tvec[GSZ + jj], tvec[jj])
            hb = pl.multiple_of((hs // 8) * 8, 8)
            pltpu.async_copy(ent_hbm.at[0, pl.ds(hb, 8), :],
                             sh.at[pl.ds(rb, 8), :], sem)
            cb = pl.multiple_of((rs // 8) * 8, 8)
            pltpu.async_copy(rel_hbm.at[0, pl.ds(cb, 8), :],
                             sr.at[pl.ds(rb, 8), :], sem)
            tb = pl.multiple_of((ts // 8) * 8, 8)
            pltpu.async_copy(ent_hbm.at[0, pl.ds(tb, 8), :],
                             st.at[pl.ds(rb, 8), :], sem)

    def wait_group(par):
        sem = sems[par]
        pltpu.make_async_copy(ent_hbm.at[0, pl.ds(0, 192), :],
                              sh.at[pl.ds(0, 192), :], sem).wait()

    def compute_group(g, par):
        vbase = pl.multiple_of((g // 2) * 16, 16)
        hvec = hvm[pl.ds(vbase, 16)]
        rvec = rvm[pl.ds(vbase, 16)]
        tvec = tvm[pl.ds(vbase, 16)]
        odd = lax.rem(g, 2)
        for jj in range(GSZ):
            rb = (par * GSZ + jj) * 8
            hs = lax.select(odd == 1, hvec[GSZ + jj], hvec[jj])
            rs = lax.select(odd == 1, rvec[GSZ + jj], rvec[jj])
            ts = lax.select(odd == 1, tvec[GSZ + jj], tvec[jj])
            hr = rb + lax.rem(hs, 8)
            rr = rb + lax.rem(rs, 8)
            tr = rb + lax.rem(ts, 8)
            acc = jnp.zeros((16,), jnp.float32)
            for k in range(D // 16):
                hv = sh[hr, pl.ds(k * 16, 16)]
                rv = sr[rr, pl.ds(k * 16, 16)]
                tv = st[tr, pl.ds(k * 16, 16)]
                acc = acc + jnp.abs(hv + rv - tv)
            plsc.store_scatter(tmp, [lanes * 16 + odd * GSZ + jj], acc)
        @pl.when(lax.rem(g, 2) == 1)
        def _():
            colsum = jnp.zeros((16,), jnp.float32)
            for l in range(16):
                colsum = colsum + tmp[pl.ds(l * 16, 16)]
            outv[pl.ds((g // 2) * 16, 16)] = -colsum

    for p in range(NP):
        issue_group(p, p)

    def step(gg, carry):
        for par in range(NP):
            g = gg * NP + par
            wait_group(par)
            compute_group(g, par)
            issue_group(g + NP, par)
        return carry

    lax.fori_loop(0, NG // NP - 1, step, 0)
    for par in range(NP):
        g = NG - NP + par
        wait_group(par)
        compute_group(g, par)

    pltpu.sync_copy(outv, out_hbm.at[pl.ds(base, BW)])


def kernel(entity_table, relation_table, heads, relations, tails):
    mesh = plsc.VectorSubcoreMesh(core_axis_name="c", subcore_axis_name="s")
    run = functools.partial(
        pl.kernel,
        mesh=mesh,
        compiler_params=pltpu.CompilerParams(
            needs_layout_passes=False, use_tc_tiling_on_sc=True),
        out_type=jax.ShapeDtypeStruct((B,), jnp.float32),
        scratch_types=[
            pltpu.VMEM((BW,), jnp.int32),
            pltpu.VMEM((BW,), jnp.int32),
            pltpu.VMEM((BW,), jnp.int32),
            pltpu.VMEM((256, D), jnp.float32),
            pltpu.VMEM((256, D), jnp.float32),
            pltpu.VMEM((256, D), jnp.float32),
            pltpu.VMEM((BW,), jnp.float32),
            pltpu.VMEM((256,), jnp.float32),
            pltpu.SemaphoreType.DMA,
            pltpu.SemaphoreType.DMA,
            pltpu.SemaphoreType.DMA,
            pltpu.SemaphoreType.DMA,
        ],
    )(_transe_body)
    e3 = entity_table.reshape(1, -1, D)
    r3 = relation_table.reshape(1, -1, D)
    return run(e3, r3, heads, relations, tails)


# speedup vs baseline: 1.0005x; 1.0005x over previous
import functools

import jax
import jax.numpy as jnp
from jax import lax
from jax.experimental import pallas as pl
from jax.experimental.pallas import tpu as pltpu
from jax.experimental.pallas import tpu_sc as plsc

B = 16384
D = 64
NC = 2
NS = 16
NW = NC * NS
BW = B // NW
NG = BW // 16


def _transe_body(ent_hbm, rel_hbm, heads_hbm, rels_hbm, tails_hbm, out_hbm,
                 hvm, rvm, tvm, sh, sr, st, outv, tmp, sem0, sem1):
    wid = lax.axis_index("s") * NC + lax.axis_index("c")
    base = wid * BW

    pltpu.sync_copy(heads_hbm.at[pl.ds(base, BW)], hvm)
    pltpu.sync_copy(rels_hbm.at[pl.ds(base, BW)], rvm)
    pltpu.sync_copy(tails_hbm.at[pl.ds(base, BW)], tvm)

    sems = (sem0, sem1)
    lanes = lax.iota(jnp.int32, 16)

    def issue_group(g, par):
        sem = sems[par]
        hvec = hvm[pl.ds(g * 16, 16)]
        rvec = rvm[pl.ds(g * 16, 16)]
        tvec = tvm[pl.ds(g * 16, 16)]
        for jj in range(16):
            rb = (par * 16 + jj) * 8
            hb = pl.multiple_of((hvec[jj] // 8) * 8, 8)
            pltpu.async_copy(ent_hbm.at[0, pl.ds(hb, 8), :],
                             sh.at[pl.ds(rb, 8), :], sem)
            cb = pl.multiple_of((rvec[jj] // 8) * 8, 8)
            pltpu.async_copy(rel_hbm.at[0, pl.ds(cb, 8), :],
                             sr.at[pl.ds(rb, 8), :], sem)
            tb = pl.multiple_of((tvec[jj] // 8) * 8, 8)
            pltpu.async_copy(ent_hbm.at[0, pl.ds(tb, 8), :],
                             st.at[pl.ds(rb, 8), :], sem)

    def wait_group(par):
        sem = sems[par]
        pltpu.make_async_copy(ent_hbm.at[0, pl.ds(0, 256), :],
                              sh, sem).wait()
        pltpu.make_async_copy(ent_hbm.at[0, pl.ds(0, 128), :],
                              sh.at[pl.ds(0, 128), :], sem).wait()

    def compute_group(g, par):
        hvec = hvm[pl.ds(g * 16, 16)]
        rvec = rvm[pl.ds(g * 16, 16)]
        tvec = tvm[pl.ds(g * 16, 16)]
        for jj in range(16):
            rb = (par * 16 + jj) * 8
            hr = rb + lax.rem(hvec[jj], 8)
            rr = rb + lax.rem(rvec[jj], 8)
            tr = rb + lax.rem(tvec[jj], 8)
            acc = jnp.zeros((16,), jnp.float32)
            for k in range(D // 16):
                hv = sh[hr, pl.ds(k * 16, 16)]
                rv = sr[rr, pl.ds(k * 16, 16)]
                tv = st[tr, pl.ds(k * 16, 16)]
                acc = acc + jnp.abs(hv + rv - tv)
            plsc.store_scatter(tmp, [lanes * 16 + jj], acc)
        colsum = jnp.zeros((16,), jnp.float32)
        for l in range(16):
            colsum = colsum + tmp[pl.ds(l * 16, 16)]
        outv[pl.ds(g * 16, 16)] = -colsum

    issue_group(0, 0)
    issue_group(1, 1)

    def step(gg, carry):
        for par in range(2):
            g = gg * 2 + par
            wait_group(par)
            compute_group(g, par)
            issue_group(g + 2, par)
        return carry

    lax.fori_loop(0, NG // 2 - 1, step, 0)
    for par in range(2):
        g = NG - 2 + par
        wait_group(par)
        compute_group(g, par)

    pltpu.sync_copy(outv, out_hbm.at[pl.ds(base, BW)])


def kernel(entity_table, relation_table, heads, relations, tails):
    mesh = plsc.VectorSubcoreMesh(core_axis_name="c", subcore_axis_name="s")
    run = functools.partial(
        pl.kernel,
        mesh=mesh,
        compiler_params=pltpu.CompilerParams(
            needs_layout_passes=False, use_tc_tiling_on_sc=True),
        out_type=jax.ShapeDtypeStruct((B,), jnp.float32),
        scratch_types=[
            pltpu.VMEM((BW,), jnp.int32),
            pltpu.VMEM((BW,), jnp.int32),
            pltpu.VMEM((BW,), jnp.int32),
            pltpu.VMEM((256, D), jnp.float32),
            pltpu.VMEM((256, D), jnp.float32),
            pltpu.VMEM((256, D), jnp.float32),
            pltpu.VMEM((BW,), jnp.float32),
            pltpu.VMEM((256,), jnp.float32),
            pltpu.SemaphoreType.DMA,
            pltpu.SemaphoreType.DMA,
        ],
    )(_transe_body)
    e3 = entity_table.reshape(1, -1, D)
    r3 = relation_table.reshape(1, -1, D)
    return run(e3, r3, heads, relations, tails)


# speedup vs baseline: 1.0030x; 1.0025x over previous
import functools

import jax
import jax.numpy as jnp
from jax import lax
from jax.experimental import pallas as pl
from jax.experimental.pallas import tpu as pltpu
from jax.experimental.pallas import tpu_sc as plsc

B = 16384
D = 64
NC = 2
NS = 16
NW = NC * NS
BW = B // NW
NG = BW // 16


def _transe_body(ent_hbm, rel_hbm, heads_hbm, rels_hbm, tails_hbm, out_hbm,
                 hvm, rvm, tvm, sh, sr, st, outv, tmp, sem0, sem1):
    wid = lax.axis_index("s") * NC + lax.axis_index("c")
    base = wid * BW

    pltpu.sync_copy(heads_hbm.at[pl.ds(base, BW)], hvm)
    pltpu.sync_copy(rels_hbm.at[pl.ds(base, BW)], rvm)
    pltpu.sync_copy(tails_hbm.at[pl.ds(base, BW)], tvm)

    sems = (sem0, sem1)
    lanes = lax.iota(jnp.int32, 16)

    def issue_group(g, par):
        sem = sems[par]
        hvec = hvm[pl.ds(g * 16, 16)]
        rvec = rvm[pl.ds(g * 16, 16)]
        tvec = tvm[pl.ds(g * 16, 16)]
        for jj in range(16):
            rb = (par * 16 + jj) * 8
            hb = pl.multiple_of((hvec[jj] // 8) * 8, 8)
            pltpu.async_copy(ent_hbm.at[0, pl.ds(hb, 8), :],
                             sh.at[pl.ds(rb, 8), :], sem)
            cb = pl.multiple_of((rvec[jj] // 8) * 8, 8)
            pltpu.async_copy(rel_hbm.at[0, pl.ds(cb, 8), :],
                             sr.at[pl.ds(rb, 8), :], sem)
            tb = pl.multiple_of((tvec[jj] // 8) * 8, 8)
            pltpu.async_copy(ent_hbm.at[0, pl.ds(tb, 8), :],
                             st.at[pl.ds(rb, 8), :], sem)

    def wait_group(par):
        sem = sems[par]
        pltpu.make_async_copy(ent_hbm.at[0, pl.ds(0, 256), :],
                              sh, sem).wait()
        pltpu.make_async_copy(ent_hbm.at[0, pl.ds(0, 128), :],
                              sh.at[pl.ds(0, 128), :], sem).wait()

    def compute_group(g, par):
        hvec = hvm[pl.ds(g * 16, 16)]
        rvec = rvm[pl.ds(g * 16, 16)]
        tvec = tvm[pl.ds(g * 16, 16)]
        for jj in range(16):
            rb = (par * 16 + jj) * 8
            hr = rb + lax.rem(hvec[jj], 8)
            rr = rb + lax.rem(rvec[jj], 8)
            tr = rb + lax.rem(tvec[jj], 8)
            acc = jnp.zeros((16,), jnp.float32)
            for k in range(D // 16):
                hv = sh[hr, pl.ds(k * 16, 16)]
                rv = sr[rr, pl.ds(k * 16, 16)]
                tv = st[tr, pl.ds(k * 16, 16)]
                acc = acc + jnp.abs(hv + rv - tv)
            plsc.store_scatter(tmp, [lanes * 16 + jj], acc)
        colsum = jnp.zeros((16,), jnp.float32)
        for l in range(16):
            colsum = colsum + tmp[pl.ds(l * 16, 16)]
        outv[pl.ds(g * 16, 16)] = -colsum

    issue_group(0, 0)
    issue_group(1, 1)

    def step(gg, carry):
        for par in range(2):
            g = gg * 2 + par
            wait_group(par)
            compute_group(g, par)
            issue_group(g + 2, par)
        return carry

    lax.fori_loop(0, NG // 2 - 1, step, 0)
    for par in range(2):
        g = NG - 2 + par
        wait_group(par)
        compute_group(g, par)

    pltpu.sync_copy(outv, out_hbm.at[pl.ds(base, BW)])


def kernel(entity_table, relation_table, heads, relations, tails):
    mesh = plsc.VectorSubcoreMesh(core_axis_name="c", subcore_axis_name="s")
    run = functools.partial(
        pl.kernel,
        mesh=mesh,
        compiler_params=pltpu.CompilerParams(
            needs_layout_passes=False, use_tc_tiling_on_sc=True,
            disable_bounds_checks=True),
        out_type=jax.ShapeDtypeStruct((B,), jnp.float32),
        scratch_types=[
            pltpu.VMEM((BW,), jnp.int32),
            pltpu.VMEM((BW,), jnp.int32),
            pltpu.VMEM((BW,), jnp.int32),
            pltpu.VMEM((256, D), jnp.float32),
            pltpu.VMEM((256, D), jnp.float32),
            pltpu.VMEM((256, D), jnp.float32),
            pltpu.VMEM((BW,), jnp.float32),
            pltpu.VMEM((256,), jnp.float32),
            pltpu.SemaphoreType.DMA,
            pltpu.SemaphoreType.DMA,
        ],
    )(_transe_body)
    e3 = entity_table.reshape(1, -1, D)
    r3 = relation_table.reshape(1, -1, D)
    return run(e3, r3, heads, relations, tails)
